# Initial kernel scaffold; baseline (speedup 1.0000x reference)
import jax
import jax.numpy as jnp
from jax.experimental import pallas as pl


def kernel(spikes, coords, conv_w, conv_b, fc_w, fc_b, Wloc, bloc, W1, b1, W2, b2, W3, b3, W_logits, W_weight):
    raise NotImplementedError("write your pallas kernel here")


# speedup vs baseline: 15.1817x; 15.1817x over previous
import functools
import jax
import jax.numpy as jnp
from jax import lax
from jax.experimental import pallas as pl
from jax.experimental.pallas import tpu as pltpu
from jax.experimental.pallas import tpu_sc as plsc

F32 = jnp.float32
N = 2048
D = 128
T = 128
TH = 64
K = 16
W = 2 * K + 1
NPAD = N + 2 * K
CSPAD = 2176
MROWS = 40


def _lift_body(x_ref, wb_ref, bv_ref, p_ref, fcw_ref, fcb_ref, out_ref):
    x = x_ref[...]
    R = x.shape[0]
    z = jnp.zeros((R, 2), F32)
    xp = jnp.concatenate([z, x, z], axis=1)
    conv = jnp.dot(xp, wb_ref[...], preferred_element_type=F32)
    conv = jnp.maximum(conv + bv_ref[...], 0.0)
    m = jnp.dot(conv, p_ref[...], preferred_element_type=F32)
    h = jnp.dot(m, fcw_ref[...], preferred_element_type=F32)
    out_ref[...] = jnp.maximum(h + fcb_ref[...], 0.0)


def _lift(spikes2d, w_band, b_vec, p_mean, fc_w, fc_b):
    BN = spikes2d.shape[0]
    R = 128
    return pl.pallas_call(
        _lift_body,
        grid=(BN // R,),
        in_specs=[
            pl.BlockSpec((R, T), lambda i: (i, 0)),
            pl.BlockSpec((T + 4, TH * T), lambda i: (0, 0)),
            pl.BlockSpec((1, TH * T), lambda i: (0, 0)),
            pl.BlockSpec((TH * T, TH), lambda i: (0, 0)),
            pl.BlockSpec((TH, D), lambda i: (0, 0)),
            pl.BlockSpec((1, D), lambda i: (0, 0)),
        ],
        out_specs=pl.BlockSpec((R, D), lambda i: (i, 0)),
        out_shape=jax.ShapeDtypeStruct((BN, D), F32),
    )(spikes2d, w_band, b_vec, p_mean, fc_w, fc_b)


def _graph_body(cf_ref, r_ref, inv_ref, cs_ref, mask_ref):
    BI = 256
    cv = cf_ref[0]
    cjT = jnp.transpose(cv)
    jj = lax.broadcasted_iota(jnp.int32, (N, BI), 0).astype(F32)
    iow = lax.broadcasted_iota(jnp.int32, (N, BI), 1).astype(F32)
    rparts = []
    for k in range(N // BI):
        ci = cv[:, k * BI:(k + 1) * BI]
        ii = iow + (k * BI)
        less = (cjT < ci) | ((cjT == ci) & (jj < ii))
        rparts.append(jnp.sum(less.astype(F32), axis=0, keepdims=True))
    rv = jnp.concatenate(rparts, axis=1)
    r_ref[0] = rv
    rT = jnp.transpose(rv)
    iparts = []
    cparts = []
    for k in range(N // BI):
        pv = iow + (k * BI)
        oh = (rT == pv).astype(F32)
        iparts.append(jnp.sum(oh * jj, axis=0, keepdims=True))
        cparts.append(jnp.sum(oh * cjT, axis=0, keepdims=True))
    invv = jnp.concatenate(iparts, axis=1)
    csv = jnp.concatenate(cparts, axis=1)
    inv_ref[0] = invv
    cs_ref[0] = csv
    big = jnp.full((1, K), 1e30, F32)
    bigo = jnp.full((1, K), 1e9, F32)
    csp = jnp.concatenate([big, csv, big], axis=1)
    invp = jnp.concatenate([bigo, invv, bigo], axis=1)
    ds = []
    os_ = []
    for a in range(W):
        da = csv - csp[:, a:a + N]
        ds.append(da * da)
        os_.append(invp[:, a:a + N])
    dmat = jnp.concatenate(ds, axis=0)
    omat = jnp.concatenate(os_, axis=0)
    cnt = jnp.zeros((W, N), F32)
    for b in range(W):
        db = dmat[b:b + 1, :]
        ob = omat[b:b + 1, :]
        beats = (db < dmat) | ((db == dmat) & (ob < omat))
        cnt = cnt + beats.astype(F32)
    sel = (cnt < float(K)).astype(F32)
    mask_ref[...] = jnp.concatenate(
        [sel, jnp.zeros((MROWS - W, N), F32)], axis=0)[None]


def _graph(c3):
    B = c3.shape[0]
    return pl.pallas_call(
        _graph_body,
        grid=(B,),
        in_specs=[pl.BlockSpec((1, 1, N), lambda b: (b, 0, 0))],
        out_specs=[
            pl.BlockSpec((1, 1, N), lambda b: (b, 0, 0)),
            pl.BlockSpec((1, 1, N), lambda b: (b, 0, 0)),
            pl.BlockSpec((1, 1, N), lambda b: (b, 0, 0)),
            pl.BlockSpec((1, MROWS, N), lambda b: (b, 0, 0)),
        ],
        out_shape=[
            jax.ShapeDtypeStruct((B, 1, N), F32),
            jax.ShapeDtypeStruct((B, 1, N), F32),
            jax.ShapeDtypeStruct((B, 1, N), F32),
            jax.ShapeDtypeStruct((B, MROWS, N), F32),
        ],
    )(c3)


def _sc_gather(table, idx):
    rows, d = table.shape
    info = plsc.get_sparse_core_info()
    nw = info.num_cores * info.num_subcores
    bpw = rows // nw
    mesh = plsc.VectorSubcoreMesh(core_axis_name="c", subcore_axis_name="s")

    @functools.partial(
        pl.kernel,
        mesh=mesh,
        out_type=jax.ShapeDtypeStruct((rows, d), F32),
        scratch_types=[
            pltpu.VMEM((bpw,), jnp.int32),
            pltpu.VMEM((bpw, d), F32),
            pltpu.SemaphoreType.DMA,
        ],
    )
    def gk(table_hbm, idx_hbm, out_hbm, idx_v, rows_v, sem):
        wid = lax.axis_index("s") * info.num_cores + lax.axis_index("c")
        base = wid * bpw
        pltpu.sync_copy(idx_hbm.at[pl.ds(base, bpw)], idx_v)
        pltpu.async_copy(table_hbm.at[idx_v], rows_v, sem).wait()
        pltpu.sync_copy(rows_v, out_hbm.at[pl.ds(base, bpw)])

    return gk(table, idx)


def _layers_body(us0_ref, cs_ref, mask_ref, w1a_ref, w1b_ref, w2_ref,
                 w3p_ref, wloc_ref, vecs_ref, out_ref, scr_ref):
    l = pl.program_id(0)
    b = pl.program_id(1)
    i = pl.program_id(2)
    rb = out_ref.shape[1]
    base = i * rb
    src = lax.rem(l, 2)
    dst = 1 - src

    @pl.when((l == 0) & (i == 0))
    def _():
        scr_ref[0, b] = us0_ref[0]
        scr_ref[1, b, 0:K, :] = jnp.zeros((K, D), F32)
        scr_ref[1, b, NPAD - K:NPAD, :] = jnp.zeros((K, D), F32)

    usw = scr_ref[src, b, pl.ds(base, rb + 2 * K), :]
    csw = jnp.transpose(cs_ref[0, :, pl.ds(base, rb + 128)])[:rb + 2 * K]
    usb = usw[K:K + rb]
    csb = csw[K:K + rb]
    v = vecs_ref[0]
    w1xi = v[0:1]
    w1xj = v[1:2]
    b1 = v[2:3]
    b2 = v[3:4]
    bloc = v[5:6]
    b3 = v[6:7, 0:1]
    a_mat = jnp.dot(usb, w1a_ref[0], preferred_element_type=F32) \
        + csb * w1xi + b1
    bt = jnp.dot(usw, w1b_ref[0], preferred_element_type=F32) \
        + csw * w1xj
    maskb = jnp.transpose(mask_ref[0])
    w2 = w2_ref[0]
    w3p = w3p_ref[0]
    h1_all = jnp.concatenate([a_mat + bt[a:a + rb] for a in range(W)],
                             axis=0)
    h1_all = jnp.maximum(h1_all, 0.0)
    h2_all = jnp.maximum(
        jnp.dot(h1_all, w2, preferred_element_type=F32) + b2, 0.0)
    kv_all = jnp.dot(h2_all, w3p, preferred_element_type=F32)[:, 0:1] + b3
    agg = jnp.zeros((rb, D), F32)
    for a in range(W):
        km = kv_all[a * rb:(a + 1) * rb] * maskb[:, a:a + 1] * 0.25
        agg = agg + km * usw[a:a + rb]
    out = jnp.dot(usb, wloc_ref[0], preferred_element_type=F32) \
        + bloc + agg + usb
    res = jnp.maximum(out, 0.0)
    scr_ref[dst, b, pl.ds(base + K, rb), :] = res
    out_ref[0] = res


def _layers(us_pad, csf, mask, w1a_all, w1b_all, w2_all, w3p_all,
            wloc_all, vecs_all):
    B = us_pad.shape[0]
    L = w2_all.shape[0]
    RB = 256
    return pl.pallas_call(
        _layers_body,
        grid=(L, B, N // RB),
        in_specs=[
            pl.BlockSpec((1, NPAD, D), lambda l, b, i: (b, 0, 0)),
            pl.BlockSpec((1, 1, CSPAD), lambda l, b, i: (b, 0, 0)),
            pl.BlockSpec((1, MROWS, RB), lambda l, b, i: (b, 0, i)),
            pl.BlockSpec((1, D, D), lambda l, b, i: (l, 0, 0)),
            pl.BlockSpec((1, D, D), lambda l, b, i: (l, 0, 0)),
            pl.BlockSpec((1, D, D), lambda l, b, i: (l, 0, 0)),
            pl.BlockSpec((1, D, D), lambda l, b, i: (l, 0, 0)),
            pl.BlockSpec((1, D, D), lambda l, b, i: (l, 0, 0)),
            pl.BlockSpec((1, 8, D), lambda l, b, i: (l, 0, 0)),
        ],
        out_specs=pl.BlockSpec((1, RB, D), lambda l, b, i: (b, i, 0)),
        out_shape=jax.ShapeDtypeStruct((B, N, D), F32),
        scratch_shapes=[pltpu.VMEM((2, B, NPAD, D), F32)],
    )(us_pad, csf, mask, w1a_all, w1b_all, w2_all, w3p_all,
      wloc_all, vecs_all)


def _dec_body(hb_ref, hf_ref, wl_ref, ww_ref, lo_ref, pr_ref, we_ref):
    hb = hb_ref[0]
    hf = hf_ref[0]
    nt = (((1,), (1,)), ((), ()))
    l1 = jnp.dot(hb, wl_ref[...], preferred_element_type=F32)
    logits = lax.dot_general(l1, hf, nt, preferred_element_type=F32)
    w1 = jnp.dot(hb, ww_ref[...], preferred_element_type=F32)
    wraw = lax.dot_general(w1, hf, nt, preferred_element_type=F32)
    lo_ref[0] = logits
    pr_ref[0] = 1.0 / (1.0 + jnp.exp(-logits))
    we_ref[0] = jnp.maximum(wraw, 0.0) + jnp.log1p(jnp.exp(-jnp.abs(wraw)))


def _decoder(hf, w_logits, w_weight):
    B = hf.shape[0]
    return pl.pallas_call(
        _dec_body,
        grid=(B, N // 128),
        in_specs=[
            pl.BlockSpec((1, 128, D), lambda b, i: (b, i, 0)),
            pl.BlockSpec((1, N, D), lambda b, i: (b, 0, 0)),
            pl.BlockSpec((D, D), lambda b, i: (0, 0)),
            pl.BlockSpec((D, D), lambda b, i: (0, 0)),
        ],
        out_specs=[
            pl.BlockSpec((1, 128, N), lambda b, i: (b, i, 0)),
            pl.BlockSpec((1, 128, N), lambda b, i: (b, i, 0)),
            pl.BlockSpec((1, 128, N), lambda b, i: (b, i, 0)),
        ],
        out_shape=[
            jax.ShapeDtypeStruct((B, N, N), F32),
            jax.ShapeDtypeStruct((B, N, N), F32),
            jax.ShapeDtypeStruct((B, N, N), F32),
        ],
    )(hf, hf, w_logits, w_weight)


def kernel(spikes, coords, conv_w, conv_b, fc_w, fc_b, Wloc, bloc,
           W1, b1, W2, b2, W3, b3, W_logits, W_weight):
    B = spikes.shape[0]
    L = Wloc.shape[0]
    c3 = coords[..., 0].reshape(B, 1, N)

    eyes = jnp.stack([jnp.eye(T + 4, T, k=-s, dtype=F32) for s in range(5)])
    w_band = jnp.einsum('sit,os->iot', eyes,
                        conv_w[:, 0, :]).reshape(T + 4, TH * T)
    b_vec = jnp.repeat(conv_b, T).reshape(1, TH * T)
    p_mean = (jnp.repeat(jnp.eye(TH, dtype=F32), T, axis=0) / T)

    h0 = _lift(spikes.reshape(B * N, T), w_band, b_vec, p_mean,
               fc_w, fc_b.reshape(1, D))

    r, inv, cs, mask = _graph(c3)

    pad3 = ((0, 0), (0, 0), (K, CSPAD - N - K))
    boff = (jnp.arange(B, dtype=jnp.int32) * N)[:, None, None]
    idx0 = (inv.astype(jnp.int32) + boff).reshape(-1)
    us = _sc_gather(h0, idx0).reshape(B, N, D)

    csf = jnp.pad(cs, pad3)
    us_pad = jnp.pad(us, ((0, 0), (K, K), (0, 0)))
    w1a_all = W1[:, 2:2 + D]
    w1b_all = W1[:, 2 + D:2 + 2 * D]
    w3p_all = jnp.concatenate([W3, jnp.zeros((L, D, D - 1), F32)], axis=2)
    vecs_all = jnp.stack([
        W1[:, 0], W1[:, 1], b1, b2, W3[:, :, 0], bloc,
        jnp.broadcast_to(b3, (L, D)), jnp.zeros((L, D), F32)], axis=1)
    us = _layers(us_pad, csf, mask, w1a_all, w1b_all, W2, w3p_all,
                 Wloc, vecs_all)

    idx1 = (r.astype(jnp.int32) + boff).reshape(-1)
    hf = _sc_gather(us.reshape(B * N, D), idx1).reshape(B, N, D)

    return _decoder(hf, W_logits, W_weight)
